# baseline (speedup 1.0000x reference)
import functools
import math

import jax
import jax.numpy as jnp
from jax import lax
from jax.experimental import pallas as pl
from jax.experimental.pallas import tpu as pltpu
from jax.experimental.pallas import tpu_sc as plsc

D_MODEL = 256
HALF = D_MODEL // 2
HEIGHT = 20.0
WIDTH = 32.0
N_SPECIAL = 3

_V_PAD = 648
_NC, _NS = 2, 16
_NW = _NC * _NS
_B = 1024 * 200
_NG = _NW // 2
_TPG = _B // _NG
_CH = 64
_NCH = _TPG // _CH
_L = 16


def _table_kernel(gauss_ref, tab_ref):
    v = lax.broadcasted_iota(jnp.int32, (_V_PAD, 1), 0)
    valid = v >= 2
    a = (v - N_SPECIAL).astype(jnp.float32)
    q = jnp.floor(a / WIDTH)
    xf = a - WIDTH * q
    cx = 2.0 * (xf / WIDTH) - 1.0
    cy = 2.0 * (q / HEIGHT) - 1.0
    cxb = cx.astype(jnp.bfloat16).astype(jnp.float32)
    cyb = cy.astype(jnp.bfloat16).astype(jnp.float32)
    g0 = gauss_ref[0:1, :].astype(jnp.bfloat16).astype(jnp.float32)
    g1 = gauss_ref[1:2, :].astype(jnp.bfloat16).astype(jnp.float32)
    t = cxb * g0 + cyb * g1
    f = (2.0 * math.pi) * (t - jnp.round(t))
    pe = jnp.concatenate([jnp.sin(f), jnp.cos(f)], axis=-1)
    tab_ref[...] = jnp.where(valid, pe, 0.0)


def _build_table(gauss):
    return pl.pallas_call(
        _table_kernel,
        out_shape=jax.ShapeDtypeStruct((_V_PAD, D_MODEL), jnp.float32),
    )(gauss)


@functools.lru_cache(maxsize=1)
def _make_sc_expand():
    mesh = plsc.VectorSubcoreMesh(core_axis_name="c", subcore_axis_name="s")

    @functools.partial(
        pl.kernel,
        out_type=jax.ShapeDtypeStruct((_B, D_MODEL), jnp.float32),
        mesh=mesh,
        scratch_types=[
            pltpu.VMEM((_V_PAD * HALF,), jnp.float32),
            pltpu.VMEM_SHARED((_NG, _TPG), jnp.int32),
            pltpu.VMEM((_CH, HALF), jnp.float32),
            pltpu.VMEM((_CH, HALF), jnp.float32),
            pltpu.SMEM((_CH,), jnp.int32),
            pltpu.SMEM((_CH,), jnp.int32),
            pltpu.SemaphoreType.DMA,
            pltpu.SemaphoreType.DMA,
        ],
    )
    def _sc_expand(
        tab0_hbm, tab1_hbm, idx_hbm, out_hbm,
        tab_v, idx_sp, buf0, buf1, sm0, sm1, w0, w1,
    ):
        sid = lax.axis_index("s")
        h = lax.axis_index("c")
        g = sid

        @pl.when(h == 0)
        def _():
            pltpu.sync_copy(tab0_hbm, tab_v)

        @pl.when(h == 1)
        def _():
            pltpu.sync_copy(tab1_hbm, tab_v)

        @pl.when(sid == 0)
        def _():
            pltpu.sync_copy(idx_hbm, idx_sp)

        plsc.subcore_barrier()

        bufs = (buf0, buf1)
        sms = (sm0, sm1)
        wsems = (w0, w1)

        def _fill(i, b):
            buf = bufs[b]
            sm = sms[b]
            pltpu.sync_copy(idx_sp.at[g, pl.ds(i * _CH, _CH)], sm)
            for t in range(_CH):
                rowb = sm[t] * HALF
                for c in range(HALF // _L):
                    buf[t, pl.ds(_L * c, _L)] = tab_v[pl.ds(rowb + _L * c, _L)]

        def _write(i, b):
            pltpu.async_copy(
                bufs[b],
                out_hbm.at[pl.ds(g * _TPG + i * _CH, _CH),
                           pl.ds(h * HALF, HALF)],
                wsems[b],
            )

        def _wait_write(i, b):
            pltpu.make_async_copy(
                bufs[b],
                out_hbm.at[pl.ds(g * _TPG + i * _CH, _CH),
                           pl.ds(h * HALF, HALF)],
                wsems[b],
            ).wait()

        def pair(j, _):
            i0 = 2 * j
            for b in range(2):
                i = i0 + b

                @pl.when(i >= 2)
                def _():
                    _wait_write(i - 2, b)

                _fill(i, b)
                _write(i, b)
            return 0

        lax.fori_loop(0, _NCH // 2, pair, 0)
        _wait_write(_NCH - 2, 0)
        _wait_write(_NCH - 1, 1)

    return _sc_expand


@jax.jit
def kernel(tgt_seq, gauss):
    b, s = tgt_seq.shape
    table = _build_table(gauss)
    tab0 = table[:, :HALF].reshape(-1)
    tab1 = table[:, HALF:].reshape(-1)
    idx = tgt_seq.reshape(_NG, _TPG)
    out = _make_sc_expand()(tab0, tab1, idx)
    return out.reshape(b, s, D_MODEL)


# speedup vs baseline: 2.5884x; 2.5884x over previous
import functools
import math

import jax
import jax.numpy as jnp
from jax import lax
from jax.experimental import pallas as pl
from jax.experimental.pallas import tpu as pltpu
from jax.experimental.pallas import tpu_sc as plsc

D_MODEL = 256
HALF = D_MODEL // 2
HEIGHT = 20.0
WIDTH = 32.0
N_SPECIAL = 3

_V_PAD = 648
_R = 8
_NC, _NS = 2, 16
_NW = _NC * _NS
_B = 1024 * 200
_BPW = _B // _NW
_CH = 128
_NCH = _BPW // _CH
_NB = 2


def _table_kernel(gauss_ref, tab_ref):
    v = lax.broadcasted_iota(jnp.int32, (_V_PAD, 1), 0)
    valid = v >= 2
    a = (v - N_SPECIAL).astype(jnp.float32)
    q = jnp.floor(a / WIDTH)
    xf = a - WIDTH * q
    cx = 2.0 * (xf / WIDTH) - 1.0
    cy = 2.0 * (q / HEIGHT) - 1.0
    cxb = cx.astype(jnp.bfloat16).astype(jnp.float32)
    cyb = cy.astype(jnp.bfloat16).astype(jnp.float32)
    g0 = gauss_ref[0:1, :].astype(jnp.bfloat16).astype(jnp.float32)
    g1 = gauss_ref[1:2, :].astype(jnp.bfloat16).astype(jnp.float32)
    t = cxb * g0 + cyb * g1
    f = (2.0 * math.pi) * (t - jnp.round(t))
    pe = jnp.concatenate([jnp.sin(f), jnp.cos(f)], axis=-1)
    tab_ref[...] = jnp.where(valid, pe, 0.0)


def _build_table(gauss):
    return pl.pallas_call(
        _table_kernel,
        grid=(_R,),
        in_specs=[pl.BlockSpec((2, HALF), lambda i: (0, 0))],
        out_specs=pl.BlockSpec((_V_PAD, D_MODEL), lambda i: (i, 0)),
        out_shape=jax.ShapeDtypeStruct((_R * _V_PAD, D_MODEL), jnp.float32),
    )(gauss)


@functools.lru_cache(maxsize=1)
def _make_sc_gather():
    mesh = plsc.VectorSubcoreMesh(core_axis_name="c", subcore_axis_name="s")

    @functools.partial(
        pl.kernel,
        out_type=jax.ShapeDtypeStruct((_B, D_MODEL), jnp.float32),
        mesh=mesh,
        scratch_types=[
            pltpu.VMEM((_NCH, _CH), jnp.int32),
        ]
        + [pltpu.VMEM((_CH, D_MODEL), jnp.float32)] * _NB
        + [pltpu.SemaphoreType.DMA] * (2 * _NB),
    )
    def _sc_gather(tab_hbm, idx_hbm, out_hbm, idx_v, *bufs_sems):
        bufs = bufs_sems[:_NB]
        gsems = bufs_sems[_NB : 2 * _NB]
        wsems = bufs_sems[2 * _NB :]
        wid = lax.axis_index("s") * _NC + lax.axis_index("c")
        base = wid * _BPW
        pltpu.sync_copy(idx_hbm.at[wid], idx_v)

        def _gather(i, b):
            pltpu.async_copy(tab_hbm.at[idx_v.at[i]], bufs[b], gsems[b])

        def _wait_gather(i, b):
            pltpu.make_async_copy(
                tab_hbm.at[idx_v.at[i]], bufs[b], gsems[b]
            ).wait()

        def _write(i, b):
            pltpu.async_copy(
                bufs[b], out_hbm.at[pl.ds(base + i * _CH, _CH)], wsems[b]
            )

        def _wait_write(i, b):
            pltpu.make_async_copy(
                bufs[b], out_hbm.at[pl.ds(base + i * _CH, _CH)], wsems[b]
            ).wait()

        _gather(0, 0)

        def pair(j, _):
            i0 = 2 * j
            for b in range(2):
                i = i0 + b

                @pl.when(i >= 1)
                def _():
                    _wait_write(i - 1, 1 - b)

                @pl.when(i + 1 < _NCH)
                def _():
                    _gather(i + 1, 1 - b)

                _wait_gather(i, b)
                _write(i, b)
            return 0

        lax.fori_loop(0, _NCH // 2, pair, 0)
        _wait_write(_NCH - 1, (_NCH - 1) % 2)

    return _sc_gather


@jax.jit
def kernel(tgt_seq, gauss):
    b, s = tgt_seq.shape
    table = _build_table(gauss)
    rep_off = (jnp.arange(_NW, dtype=jnp.int32) % _R) * _V_PAD
    idx = tgt_seq.reshape(_NW, _NCH, _CH) + rep_off[:, None, None]
    out = _make_sc_gather()(table, idx)
    return out.reshape(b, s, D_MODEL)
